# Initial kernel scaffold; baseline (speedup 1.0000x reference)
import jax
import jax.numpy as jnp
from jax.experimental import pallas as pl


def kernel(x, params):
    raise NotImplementedError("write your pallas kernel here")


# speedup vs baseline: 5.6721x; 5.6721x over previous
import functools

import jax
import jax.numpy as jnp
from jax.experimental import pallas as pl
from jax.experimental.pallas import tpu as pltpu
from jax.experimental.pallas import tpu_sc as plsc

_PART = 50
_B, _N = 8, 2048
_ENC_CFG = [(3, 256, 8, -1), (256, 256, 12, 768), (256, 512, 16, 384),
            (512, 1024, 16, 128)]
_DEC_CFG = [(1024, 1024, 1024, 16), (1024, 512, 512, 16), (512, 256, 256, 12),
            (256, _PART, 256, 8)]


def _elu(x):
    return jnp.where(x > 0, x, jnp.exp(jnp.minimum(x, 0.0)) - 1.0)


def _pb_for(p, ci):
    if ci >= 1024:
        return 64
    return 256 if p % 256 == 0 else 128


def _head_compute(rep, p3, pm, b, K, Pb, Np):
    rx, ry, rz = rep[:, 0:1], rep[:, 1:2], rep[:, 2:3]
    px, py, pz = p3[0:1, :], p3[1:2, :], p3[2:3, :]
    rep2 = rx * rx + ry * ry + rz * rz
    pts2 = px * px + py * py + pz * pz
    cross = jnp.dot(rep, p3, preferred_element_type=jnp.float32)
    d = rep2 + pts2 - 2.0 * cross
    iota = jax.lax.broadcasted_iota(jnp.int32, (Pb, Np), 1)
    big = jnp.float32(3.0e38)
    idxs, nbs = [], []
    for _ in range(K):
        mn = jnp.min(d, axis=1, keepdims=True)
        cand = jnp.where(d <= mn, iota, jnp.int32(Np))
        ij = jnp.min(cand, axis=1, keepdims=True)
        mask = iota == ij
        nb = jax.lax.dot(mask.astype(jnp.float32), pm,
                         precision=jax.lax.Precision.HIGHEST,
                         preferred_element_type=jnp.float32)
        d = jnp.where(mask, big, d)
        idxs.append(ij)
        nbs.append(nb)
    return idxs, nbs


def _lift_and_x(plk, refs, K, Pb):
    (d1w, d1b, d2w, d2b, x0w, x0b, x1w, x1b, x2w, x2b) = refs
    lift = _elu(jnp.dot(plk, d1w[...]) + d1b[...])
    lift = _elu(jnp.dot(lift, d2w[...]) + d2b[...])
    acc = None
    for k in range(K):
        t = jnp.dot(plk[k * Pb:(k + 1) * Pb, :], x0w[k])
        acc = t if acc is None else acc + t
    X = _elu(acc + x0b[...])
    X = _elu(jnp.dot(X, x1w[...]) + x1b[...])
    X = jnp.dot(X, x2w[...]) + x2b[...]
    return lift, X


def _mix_compute(X, lift_sl, fts_sl, dwl, dwf, pwl, pwf, pwb,
                 K, Pb, Cmid, Cin, dm, G, gw):
    dwlv, dwfv = dwl[...], dwf[...]
    fXl = [None] * K
    fXf = [[None] * G for _ in range(K)]
    for l in range(K):
        lv = lift_sl(l)
        fvs = fts_sl(l)
        for k in range(K):
            xkl = X[:, k * K + l:k * K + l + 1]
            tl = xkl * lv
            fXl[k] = tl if fXl[k] is None else fXl[k] + tl
            for g in range(G):
                tf = xkl * fvs[g]
                fXf[k][g] = tf if fXf[k][g] is None else fXf[k][g] + tf
    out = pwb[...]
    for m in range(dm):
        aL = None
        aF = [None] * G
        for k in range(K):
            wl = dwlv[m * K + k:m * K + k + 1, :]
            tl = fXl[k] * wl
            aL = tl if aL is None else aL + tl
            for g in range(G):
                wf = dwfv[m * K + k:m * K + k + 1, g * gw:(g + 1) * gw]
                tf = fXf[k][g] * wf
                aF[g] = tf if aF[g] is None else aF[g] + tf
        out = out + jnp.dot(aL, pwl[m * Cmid:(m + 1) * Cmid, :])
        for g in range(G):
            out = out + jnp.dot(
                aF[g], pwf[m * Cin + g * gw:m * Cin + (g + 1) * gw, :])
    return _elu(out)


def _make_head(K, Pb, Np, Cmid, KK):
    def body(rep_ref, pr_ref, pm_ref, d1w, d1b, d2w, d2b, x0w, x0b,
             x1w, x1b, x2w, x2b, idx_o, lift_o, x_o):
        b = pl.program_id(0)
        rep = rep_ref[0]
        idxs, nbs = _head_compute(rep, pr_ref[0], pm_ref[0], b, K, Pb, Np)
        idx_km = jnp.concatenate(idxs, axis=0) + b * jnp.int32(Np)
        idx_o[0, 0] = idx_km
        nb_km = jnp.concatenate(nbs, axis=0)
        rep_t = jnp.concatenate([rep] * K, axis=0)
        plk = nb_km - rep_t
        lift, X = _lift_and_x(
            plk, (d1w, d1b, d2w, d2b, x0w, x0b, x1w, x1b, x2w, x2b), K, Pb)
        lift_o[0, 0] = lift
        x_o[0] = X
    return body


def _make_mix(K, Pb, Cmid, Cin, dm, G, gw, cl, last):
    def body(*refs):
        x_ref, lift_ref = refs[0], refs[1]
        fts_refs = refs[2:2 + G]
        dwl, dwf, pwl, pwf, pwb = refs[2 + G:7 + G]
        if cl:
            fw1, fw2, fb, skip_ref, out_ref = refs[7 + G:]
        else:
            (out_ref,) = refs[7 + G:]
        X = x_ref[0]
        lift_sl = lambda l: lift_ref[0, 0, l * Pb:(l + 1) * Pb, :]
        fts_sl = lambda l: [fr[0, 0, l * Pb:(l + 1) * Pb, :]
                            for fr in fts_refs]
        out = _mix_compute(X, lift_sl, fts_sl, dwl, dwf, pwl, pwf, pwb,
                           K, Pb, Cmid, Cin, dm, G, gw)
        if cl:
            out = jnp.dot(out, fw1[...]) + jnp.dot(skip_ref[0], fw2[...]) \
                + fb[...]
            if not last:
                out = _elu(out)
        out_ref[0] = out
    return body


def _make_level1(K, Pb, Np, Cmid, KK, dm, co):
    def body(rep_ref, pr_ref, pm_ref, d1w, d1b, d2w, d2b, x0w, x0b,
             x1w, x1b, x2w, x2b, dwl, dwf, pwl, pwf, pwb, out_ref):
        b = pl.program_id(0)
        rep = rep_ref[0]
        _, nbs = _head_compute(rep, pr_ref[0], pm_ref[0], b, K, Pb, Np)
        nb_km = jnp.concatenate(nbs, axis=0)
        rep_t = jnp.concatenate([rep] * K, axis=0)
        plk = nb_km - rep_t
        lift, X = _lift_and_x(
            plk, (d1w, d1b, d2w, d2b, x0w, x0b, x1w, x1b, x2w, x2b), K, Pb)
        lift_sl = lambda l: lift[l * Pb:(l + 1) * Pb, :]
        fts_sl = lambda l: [nbs[l]]
        out_ref[0] = _mix_compute(X, lift_sl, fts_sl, dwl, dwf, pwl, pwf,
                                  pwb, K, Pb, Cmid, 3, dm, 1, 3)
    return body


def _gather_rows(table, idx_flat):
    w = 128
    M = idx_flat.shape[1]
    D = table.shape[1]
    mesh = plsc.VectorSubcoreMesh(core_axis_name="core",
                                  subcore_axis_name="subcore")

    @functools.partial(
        pl.kernel,
        out_type=jax.ShapeDtypeStruct((M, D), table.dtype),
        mesh=mesh)
    def _k(x_hbm, i_hbm, o_hbm):
        def gbody(i_vmem, o_vmem):
            pltpu.sync_copy(x_hbm.at[i_vmem.at[0]], o_vmem)

        pltpu.emit_pipeline(
            gbody,
            grid=(M // w,),
            in_specs=[pl.BlockSpec((1, w), index_map=lambda i: (0, i))],
            out_specs=[pl.BlockSpec((w, D), index_map=lambda i: (i, 0))],
            core_axis_name=("core", "subcore"),
            dimension_semantics=(pltpu.PARALLEL,),
        )(i_hbm, o_hbm)

    return _k(table, idx_flat)


def _full(shape):
    n = len(shape)
    return pl.BlockSpec(shape, lambda b, i, _n=n: (0,) * _n)


def _prep_weights(p, K, Cmid, Cin, dm, co):
    dww = p["dw_w"]
    dwl = dww[:, :Cmid, :].transpose(2, 0, 1).reshape(dm * K, Cmid)
    dwf = dww[:, Cmid:, :].transpose(2, 0, 1).reshape(dm * K, Cin)
    pw3 = p["pw_w"].reshape(Cmid + Cin, dm, co)
    pwl = pw3[:Cmid].transpose(1, 0, 2).reshape(dm * Cmid, co)
    pwf = pw3[Cmid:].transpose(1, 0, 2).reshape(dm * Cin, co)
    return dwl, dwf, pwl, pwf


def _xconv(p, rep, pts, fts, K, ci, co, fuse=None, skip_fts=None,
           last=False):
    B, P = rep.shape[0], rep.shape[1]
    Np = pts.shape[1]
    Cmid = co // 4
    dm = min(-(-co // ci), 4)
    KK = K * K
    Pb = _pb_for(P, ci)
    nPb = P // Pb
    KPb = K * Pb
    p3 = jnp.transpose(pts, (0, 2, 1))
    dwl, dwf, pwl, pwf = _prep_weights(p, K, Cmid, ci, dm, co)
    biases = dict(
        d1b=p["d1_b"].reshape(1, -1), d2b=p["d2_b"].reshape(1, -1),
        x0b=p["x0_b"].reshape(1, -1), x1b=p["x1_b"].reshape(1, -1),
        x2b=p["x2_b"].reshape(1, -1), pwb=p["pw_b"].reshape(1, -1))

    head_in = [rep, p3, pts, p["d1_w"], biases["d1b"], p["d2_w"],
               biases["d2b"], p["x0_w"], biases["x0b"], p["x1_w"],
               biases["x1b"], p["x2_w"], biases["x2b"]]
    head_specs = [
        pl.BlockSpec((1, Pb, 3), lambda b, i: (b, i, 0)),
        pl.BlockSpec((1, 3, Np), lambda b, i: (b, 0, 0)),
        pl.BlockSpec((1, Np, 3), lambda b, i: (b, 0, 0)),
    ] + [_full(a.shape) for a in head_in[3:]]

    if ci == 3:
        body = _make_level1(K, Pb, Np, Cmid, KK, dm, co)
        out = pl.pallas_call(
            body,
            grid=(B, nPb),
            in_specs=head_specs + [_full(a.shape)
                                   for a in (dwl, dwf, pwl, pwf,
                                             biases["pwb"])],
            out_specs=pl.BlockSpec((1, Pb, co), lambda b, i: (b, i, 0)),
            out_shape=jax.ShapeDtypeStruct((B, P, co), jnp.float32),
        )(*head_in, dwl, dwf, pwl, pwf, biases["pwb"])
        return out

    idx_g, lift, X = pl.pallas_call(
        _make_head(K, Pb, Np, Cmid, KK),
        grid=(B, nPb),
        in_specs=head_specs,
        out_specs=[
            pl.BlockSpec((1, 1, KPb, 1), lambda b, i: (b, i, 0, 0)),
            pl.BlockSpec((1, 1, KPb, Cmid), lambda b, i: (b, i, 0, 0)),
            pl.BlockSpec((1, Pb, KK), lambda b, i: (b, i, 0)),
        ],
        out_shape=[
            jax.ShapeDtypeStruct((B, nPb, KPb, 1), jnp.int32),
            jax.ShapeDtypeStruct((B, nPb, KPb, Cmid), jnp.float32),
            jax.ShapeDtypeStruct((B, P, KK), jnp.float32),
        ],
    )(*head_in)

    gw = min(ci, 256)
    G = ci // gw
    table = fts.reshape(B * Np, ci)
    idx_flat = idx_g.reshape(1, B * nPb * KPb)
    fts_groups = [
        _gather_rows(table[:, g * gw:(g + 1) * gw],
                     idx_flat).reshape(B, nPb, KPb, gw)
        for g in range(G)
    ]

    mix_in = ([X, lift] + fts_groups
              + [dwl, dwf, pwl, pwf, biases["pwb"]])
    mix_specs = [
        pl.BlockSpec((1, Pb, KK), lambda b, i: (b, i, 0)),
        pl.BlockSpec((1, 1, KPb, Cmid), lambda b, i: (b, i, 0, 0)),
    ] + [pl.BlockSpec((1, 1, KPb, gw), lambda b, i: (b, i, 0, 0))
         for _ in range(G)] \
      + [_full(a.shape) for a in mix_in[2 + G:]]
    if fuse is not None:
        fw1 = fuse["fuse_w"][:co]
        fw2 = fuse["fuse_w"][co:]
        fb = fuse["fuse_b"].reshape(1, -1)
        cl = fw2.shape[0]
        mix_in += [fw1, fw2, fb, skip_fts]
        mix_specs += [_full(fw1.shape), _full(fw2.shape), _full(fb.shape),
                      pl.BlockSpec((1, Pb, cl), lambda b, i: (b, i, 0))]
    else:
        cl = 0

    out = pl.pallas_call(
        _make_mix(K, Pb, Cmid, ci, dm, G, gw, cl, last),
        grid=(B, nPb),
        in_specs=mix_specs,
        out_specs=pl.BlockSpec((1, Pb, co), lambda b, i: (b, i, 0)),
        out_shape=jax.ShapeDtypeStruct((B, P, co), jnp.float32),
    )(*mix_in)
    return out


def kernel(x, params):
    pts = jnp.transpose(x, (0, 2, 1))
    cur_pts, cur_fts = pts, pts
    levels = []
    for p, (ci, co, K, P) in zip(params["enc"], _ENC_CFG):
        rep = cur_pts if P == -1 else cur_pts[:, :P]
        out = _xconv(p, rep, cur_pts, cur_fts, K, ci, co)
        levels.append((rep, out))
        cur_pts, cur_fts = rep, out
    d_pts, d_fts = levels[3]
    skips = [levels[3], levels[2], levels[1], levels[0]]
    n_dec = len(_DEC_CFG)
    for li, (p, (skip_pts, skip_fts), (ci, co, cl, K)) in enumerate(
            zip(params["dec"], skips, _DEC_CFG)):
        d_fts = _xconv(p["xconv"], skip_pts, d_pts, d_fts, K, ci, co,
                       fuse=p, skip_fts=skip_fts, last=(li == n_dec - 1))
        d_pts = skip_pts
    return jnp.transpose(d_fts, (0, 2, 1))


# speedup vs baseline: 5.7640x; 1.0162x over previous
import functools

import jax
import jax.numpy as jnp
from jax.experimental import pallas as pl
from jax.experimental.pallas import tpu as pltpu
from jax.experimental.pallas import tpu_sc as plsc

_PART = 50
_B, _N = 8, 2048
_ENC_CFG = [(3, 256, 8, -1), (256, 256, 12, 768), (256, 512, 16, 384),
            (512, 1024, 16, 128)]
_DEC_CFG = [(1024, 1024, 1024, 16), (1024, 512, 512, 16), (512, 256, 256, 12),
            (256, _PART, 256, 8)]


def _elu(x):
    return jnp.where(x > 0, x, jnp.exp(jnp.minimum(x, 0.0)) - 1.0)


def _pb_for(p, ci):
    if ci >= 1024:
        return 128
    return 256 if p % 256 == 0 else 128


def _head_compute(rep, p3, pm, b, K, Pb, Np):
    rx, ry, rz = rep[:, 0:1], rep[:, 1:2], rep[:, 2:3]
    px, py, pz = p3[0:1, :], p3[1:2, :], p3[2:3, :]
    rep2 = rx * rx + ry * ry + rz * rz
    pts2 = px * px + py * py + pz * pz
    cross = jnp.dot(rep, p3, preferred_element_type=jnp.float32)
    d = rep2 + pts2 - 2.0 * cross
    iota = jax.lax.broadcasted_iota(jnp.int32, (Pb, Np), 1)
    big = jnp.float32(3.0e38)
    idxs, nbs = [], []
    for _ in range(K):
        mn = jnp.min(d, axis=1, keepdims=True)
        cand = jnp.where(d <= mn, iota, jnp.int32(Np))
        ij = jnp.min(cand, axis=1, keepdims=True)
        mask = iota == ij
        nb = jax.lax.dot(mask.astype(jnp.float32), pm,
                         precision=jax.lax.Precision.HIGHEST,
                         preferred_element_type=jnp.float32)
        d = jnp.where(mask, big, d)
        idxs.append(ij)
        nbs.append(nb)
    return idxs, nbs


def _lift_and_x(plk, refs, K, Pb):
    (d1w, d1b, d2w, d2b, x0w, x0b, x1w, x1b, x2w, x2b) = refs
    lift = _elu(jnp.dot(plk, d1w[...]) + d1b[...])
    lift = _elu(jnp.dot(lift, d2w[...]) + d2b[...])
    acc = None
    for k in range(K):
        t = jnp.dot(plk[k * Pb:(k + 1) * Pb, :], x0w[k])
        acc = t if acc is None else acc + t
    X = _elu(acc + x0b[...])
    X = _elu(jnp.dot(X, x1w[...]) + x1b[...])
    X = jnp.dot(X, x2w[...]) + x2b[...]
    return lift, X


def _mix_compute(X, lift_sl, fts_sl, dwl, dwf, pwl, pwf, pwb,
                 K, Pb, Cmid, Cin, dm, G, gw):
    dwlv, dwfv = dwl[...], dwf[...]
    fXl = [None] * K
    fXf = [[None] * G for _ in range(K)]
    for l in range(K):
        lv = lift_sl(l)
        fvs = fts_sl(l)
        for k in range(K):
            xkl = X[:, k * K + l:k * K + l + 1]
            tl = xkl * lv
            fXl[k] = tl if fXl[k] is None else fXl[k] + tl
            for g in range(G):
                tf = xkl * fvs[g]
                fXf[k][g] = tf if fXf[k][g] is None else fXf[k][g] + tf
    out = pwb[...]
    for m in range(dm):
        aL = None
        aF = [None] * G
        for k in range(K):
            wl = dwlv[m * K + k:m * K + k + 1, :]
            tl = fXl[k] * wl
            aL = tl if aL is None else aL + tl
            for g in range(G):
                wf = dwfv[m * K + k:m * K + k + 1, g * gw:(g + 1) * gw]
                tf = fXf[k][g] * wf
                aF[g] = tf if aF[g] is None else aF[g] + tf
        out = out + jnp.dot(aL, pwl[m * Cmid:(m + 1) * Cmid, :])
        for g in range(G):
            out = out + jnp.dot(
                aF[g], pwf[m * Cin + g * gw:m * Cin + (g + 1) * gw, :])
    return _elu(out)


def _make_head(K, Pb, Np, Cmid, KK):
    def body(rep_ref, pr_ref, pm_ref, d1w, d1b, d2w, d2b, x0w, x0b,
             x1w, x1b, x2w, x2b, idx_o, lift_o, x_o):
        b = pl.program_id(0)
        rep = rep_ref[0]
        idxs, nbs = _head_compute(rep, pr_ref[0], pm_ref[0], b, K, Pb, Np)
        idx_km = jnp.concatenate(idxs, axis=0) + b * jnp.int32(Np)
        idx_o[0, 0] = idx_km
        nb_km = jnp.concatenate(nbs, axis=0)
        rep_t = jnp.concatenate([rep] * K, axis=0)
        plk = nb_km - rep_t
        lift, X = _lift_and_x(
            plk, (d1w, d1b, d2w, d2b, x0w, x0b, x1w, x1b, x2w, x2b), K, Pb)
        lift_o[0, 0] = lift
        x_o[0] = X
    return body


def _make_mix(K, Pb, Cmid, Cin, dm, G, gw, cl, last, onehot_np=0):
    nf = 2 if onehot_np else G

    def body(*refs):
        x_ref, lift_ref = refs[0], refs[1]
        fts_refs = refs[2:2 + nf]
        dwl, dwf, pwl, pwf, pwb = refs[2 + nf:7 + nf]
        if cl:
            fw1, fw2, fb, skip_ref, out_ref = refs[7 + nf:]
        else:
            (out_ref,) = refs[7 + nf:]
        X = x_ref[0]
        lift_sl = lambda l: lift_ref[0, 0, l * Pb:(l + 1) * Pb, :]
        if onehot_np:
            idx_ref, table_ref = fts_refs
            table = table_ref[0]
            b = pl.program_id(0)
            iota_n = jax.lax.broadcasted_iota(jnp.int32, (Pb, onehot_np), 1)

            def fts_sl(l):
                il = idx_ref[0, 0, l * Pb:(l + 1) * Pb, :] \
                    - b * jnp.int32(onehot_np)
                oh = (iota_n == il).astype(jnp.float32)
                return [jax.lax.dot(oh, table,
                                    precision=jax.lax.Precision.HIGHEST,
                                    preferred_element_type=jnp.float32)]
        else:
            fts_sl = lambda l: [fr[0, 0, l * Pb:(l + 1) * Pb, :]
                                for fr in fts_refs]
        out = _mix_compute(X, lift_sl, fts_sl, dwl, dwf, pwl, pwf, pwb,
                           K, Pb, Cmid, Cin, dm, G, gw)
        if cl:
            out = jnp.dot(out, fw1[...]) + jnp.dot(skip_ref[0], fw2[...]) \
                + fb[...]
            if not last:
                out = _elu(out)
        out_ref[0] = out
    return body


def _make_level1(K, Pb, Np, Cmid, KK, dm, co):
    def body(rep_ref, pr_ref, pm_ref, d1w, d1b, d2w, d2b, x0w, x0b,
             x1w, x1b, x2w, x2b, dwl, dwf, pwl, pwf, pwb, out_ref):
        b = pl.program_id(0)
        rep = rep_ref[0]
        _, nbs = _head_compute(rep, pr_ref[0], pm_ref[0], b, K, Pb, Np)
        nb_km = jnp.concatenate(nbs, axis=0)
        rep_t = jnp.concatenate([rep] * K, axis=0)
        plk = nb_km - rep_t
        lift, X = _lift_and_x(
            plk, (d1w, d1b, d2w, d2b, x0w, x0b, x1w, x1b, x2w, x2b), K, Pb)
        lift_sl = lambda l: lift[l * Pb:(l + 1) * Pb, :]
        fts_sl = lambda l: [nbs[l]]
        out_ref[0] = _mix_compute(X, lift_sl, fts_sl, dwl, dwf, pwl, pwf,
                                  pwb, K, Pb, Cmid, 3, dm, 1, 3)
    return body


def _gather_rows(table, idx_flat):
    w = 128
    M = idx_flat.shape[1]
    D = table.shape[1]
    mesh = plsc.VectorSubcoreMesh(core_axis_name="core",
                                  subcore_axis_name="subcore")

    @functools.partial(
        pl.kernel,
        out_type=jax.ShapeDtypeStruct((M, D), table.dtype),
        mesh=mesh)
    def _k(x_hbm, i_hbm, o_hbm):
        def gbody(i_vmem, o_vmem):
            pltpu.sync_copy(x_hbm.at[i_vmem.at[0]], o_vmem)

        pltpu.emit_pipeline(
            gbody,
            grid=(M // w,),
            in_specs=[pl.BlockSpec((1, w), index_map=lambda i: (0, i))],
            out_specs=[pl.BlockSpec((w, D), index_map=lambda i: (i, 0))],
            core_axis_name=("core", "subcore"),
            dimension_semantics=(pltpu.PARALLEL,),
        )(i_hbm, o_hbm)

    return _k(table, idx_flat)


def _full(shape):
    n = len(shape)
    return pl.BlockSpec(shape, lambda b, i, _n=n: (0,) * _n)


def _prep_weights(p, K, Cmid, Cin, dm, co):
    dww = p["dw_w"]
    dwl = dww[:, :Cmid, :].transpose(2, 0, 1).reshape(dm * K, Cmid)
    dwf = dww[:, Cmid:, :].transpose(2, 0, 1).reshape(dm * K, Cin)
    pw3 = p["pw_w"].reshape(Cmid + Cin, dm, co)
    pwl = pw3[:Cmid].transpose(1, 0, 2).reshape(dm * Cmid, co)
    pwf = pw3[Cmid:].transpose(1, 0, 2).reshape(dm * Cin, co)
    return dwl, dwf, pwl, pwf


def _xconv(p, rep, pts, fts, K, ci, co, fuse=None, skip_fts=None,
           last=False):
    B, P = rep.shape[0], rep.shape[1]
    Np = pts.shape[1]
    Cmid = co // 4
    dm = min(-(-co // ci), 4)
    KK = K * K
    Pb = _pb_for(P, ci)
    nPb = P // Pb
    KPb = K * Pb
    p3 = jnp.transpose(pts, (0, 2, 1))
    dwl, dwf, pwl, pwf = _prep_weights(p, K, Cmid, ci, dm, co)
    biases = dict(
        d1b=p["d1_b"].reshape(1, -1), d2b=p["d2_b"].reshape(1, -1),
        x0b=p["x0_b"].reshape(1, -1), x1b=p["x1_b"].reshape(1, -1),
        x2b=p["x2_b"].reshape(1, -1), pwb=p["pw_b"].reshape(1, -1))

    head_in = [rep, p3, pts, p["d1_w"], biases["d1b"], p["d2_w"],
               biases["d2b"], p["x0_w"], biases["x0b"], p["x1_w"],
               biases["x1b"], p["x2_w"], biases["x2b"]]
    head_specs = [
        pl.BlockSpec((1, Pb, 3), lambda b, i: (b, i, 0)),
        pl.BlockSpec((1, 3, Np), lambda b, i: (b, 0, 0)),
        pl.BlockSpec((1, Np, 3), lambda b, i: (b, 0, 0)),
    ] + [_full(a.shape) for a in head_in[3:]]

    if ci == 3:
        body = _make_level1(K, Pb, Np, Cmid, KK, dm, co)
        out = pl.pallas_call(
            body,
            grid=(B, nPb),
            in_specs=head_specs + [_full(a.shape)
                                   for a in (dwl, dwf, pwl, pwf,
                                             biases["pwb"])],
            out_specs=pl.BlockSpec((1, Pb, co), lambda b, i: (b, i, 0)),
            out_shape=jax.ShapeDtypeStruct((B, P, co), jnp.float32),
        )(*head_in, dwl, dwf, pwl, pwf, biases["pwb"])
        return out

    idx_g, lift, X = pl.pallas_call(
        _make_head(K, Pb, Np, Cmid, KK),
        grid=(B, nPb),
        in_specs=head_specs,
        out_specs=[
            pl.BlockSpec((1, 1, KPb, 1), lambda b, i: (b, i, 0, 0)),
            pl.BlockSpec((1, 1, KPb, Cmid), lambda b, i: (b, i, 0, 0)),
            pl.BlockSpec((1, Pb, KK), lambda b, i: (b, i, 0)),
        ],
        out_shape=[
            jax.ShapeDtypeStruct((B, nPb, KPb, 1), jnp.int32),
            jax.ShapeDtypeStruct((B, nPb, KPb, Cmid), jnp.float32),
            jax.ShapeDtypeStruct((B, P, KK), jnp.float32),
        ],
    )(*head_in)

    use_sc = Np >= 1024
    if use_sc:
        gw = min(ci, 256)
        G = ci // gw
        onehot_np = 0
        table = fts.reshape(B * Np, ci)
        idx_flat = idx_g.reshape(1, B * nPb * KPb)
        fts_in = [
            _gather_rows(table[:, g * gw:(g + 1) * gw],
                         idx_flat).reshape(B, nPb, KPb, gw)
            for g in range(G)
        ]
        fts_specs = [pl.BlockSpec((1, 1, KPb, gw), lambda b, i: (b, i, 0, 0))
                     for _ in range(G)]
    else:
        gw, G = ci, 1
        onehot_np = Np
        fts_in = [idx_g, fts]
        fts_specs = [
            pl.BlockSpec((1, 1, KPb, 1), lambda b, i: (b, i, 0, 0)),
            pl.BlockSpec((1, Np, ci), lambda b, i: (b, 0, 0)),
        ]

    mix_in = ([X, lift] + fts_in
              + [dwl, dwf, pwl, pwf, biases["pwb"]])
    mix_specs = [
        pl.BlockSpec((1, Pb, KK), lambda b, i: (b, i, 0)),
        pl.BlockSpec((1, 1, KPb, Cmid), lambda b, i: (b, i, 0, 0)),
    ] + fts_specs \
      + [_full(a.shape) for a in mix_in[2 + len(fts_in):]]
    if fuse is not None:
        fw1 = fuse["fuse_w"][:co]
        fw2 = fuse["fuse_w"][co:]
        fb = fuse["fuse_b"].reshape(1, -1)
        cl = fw2.shape[0]
        mix_in += [fw1, fw2, fb, skip_fts]
        mix_specs += [_full(fw1.shape), _full(fw2.shape), _full(fb.shape),
                      pl.BlockSpec((1, Pb, cl), lambda b, i: (b, i, 0))]
    else:
        cl = 0

    out = pl.pallas_call(
        _make_mix(K, Pb, Cmid, ci, dm, G, gw, cl, last, onehot_np),
        grid=(B, nPb),
        in_specs=mix_specs,
        out_specs=pl.BlockSpec((1, Pb, co), lambda b, i: (b, i, 0)),
        out_shape=jax.ShapeDtypeStruct((B, P, co), jnp.float32),
    )(*mix_in)
    return out


def kernel(x, params):
    pts = jnp.transpose(x, (0, 2, 1))
    cur_pts, cur_fts = pts, pts
    levels = []
    for p, (ci, co, K, P) in zip(params["enc"], _ENC_CFG):
        rep = cur_pts if P == -1 else cur_pts[:, :P]
        out = _xconv(p, rep, cur_pts, cur_fts, K, ci, co)
        levels.append((rep, out))
        cur_pts, cur_fts = rep, out
    d_pts, d_fts = levels[3]
    skips = [levels[3], levels[2], levels[1], levels[0]]
    n_dec = len(_DEC_CFG)
    for li, (p, (skip_pts, skip_fts), (ci, co, cl, K)) in enumerate(
            zip(params["dec"], skips, _DEC_CFG)):
        d_fts = _xconv(p["xconv"], skip_pts, d_pts, d_fts, K, ci, co,
                       fuse=p, skip_fts=skip_fts, last=(li == n_dec - 1))
        d_pts = skip_pts
    return jnp.transpose(d_fts, (0, 2, 1))


# speedup vs baseline: 7.0340x; 1.2203x over previous
import functools

import jax
import jax.numpy as jnp
from jax.experimental import pallas as pl
from jax.experimental.pallas import tpu as pltpu
from jax.experimental.pallas import tpu_sc as plsc

_PART = 50
_B, _N = 8, 2048
_ENC_CFG = [(3, 256, 8, -1), (256, 256, 12, 768), (256, 512, 16, 384),
            (512, 1024, 16, 128)]
_DEC_CFG = [(1024, 1024, 1024, 16), (1024, 512, 512, 16), (512, 256, 256, 12),
            (256, _PART, 256, 8)]


def _elu(x):
    return jnp.where(x > 0, x, jnp.exp(jnp.minimum(x, 0.0)) - 1.0)


def _pb_for(p, ci):
    if ci >= 1024:
        return 128
    return 256 if p % 256 == 0 else 128


def _head_compute(rep, p3, K, Pb, Np):
    rx, ry, rz = rep[:, 0:1], rep[:, 1:2], rep[:, 2:3]
    px, py, pz = p3[0:1, :], p3[1:2, :], p3[2:3, :]
    rep2 = rx * rx + ry * ry + rz * rz
    pts2 = px * px + py * py + pz * pz
    cross = jnp.dot(rep, p3, preferred_element_type=jnp.float32)
    d = rep2 + pts2 - 2.0 * cross
    iota = jax.lax.broadcasted_iota(jnp.int32, (Pb, Np), 1)
    big = jnp.float32(3.0e38)
    idxs, nbs = [], []
    for _ in range(K):
        mn = jnp.min(d, axis=1, keepdims=True)
        cand = jnp.where(d <= mn, iota, jnp.int32(Np))
        ij = jnp.min(cand, axis=1, keepdims=True)
        mask = iota == ij
        mf = mask.astype(jnp.float32)
        sx = jnp.sum(mf * px, axis=1, keepdims=True)
        sy = jnp.sum(mf * py, axis=1, keepdims=True)
        sz = jnp.sum(mf * pz, axis=1, keepdims=True)
        nb = jnp.concatenate([sx, sy, sz], axis=1)
        d = jnp.where(mask, big, d)
        idxs.append(ij)
        nbs.append(nb)
    return idxs, nbs


def _lift_and_x(plk, refs, K, Pb):
    (d1w, d1b, d2w, d2b, x0w, x0b, x1w, x1b, x2w, x2b) = refs
    lift = _elu(jnp.dot(plk, d1w[...]) + d1b[...])
    lift = _elu(jnp.dot(lift, d2w[...]) + d2b[...])
    acc = None
    for k in range(K):
        t = jnp.dot(plk[k * Pb:(k + 1) * Pb, :], x0w[k])
        acc = t if acc is None else acc + t
    X = _elu(acc + x0b[...])
    X = _elu(jnp.dot(X, x1w[...]) + x1b[...])
    X = jnp.dot(X, x2w[...]) + x2b[...]
    return lift, X


def _mix_compute(X, lift_sl, fts_sl, dwl, dwf, pwl, pwf, pwb,
                 K, Pb, Cmid, Cin, dm, G, gw):
    dwlv, dwfv = dwl[...], dwf[...]
    fXl = [None] * K
    fXf = [[None] * G for _ in range(K)]
    for l in range(K):
        lv = lift_sl(l)
        fvs = fts_sl(l)
        for k in range(K):
            xkl = X[:, k * K + l:k * K + l + 1]
            tl = xkl * lv
            fXl[k] = tl if fXl[k] is None else fXl[k] + tl
            for g in range(G):
                tf = xkl * fvs[g]
                fXf[k][g] = tf if fXf[k][g] is None else fXf[k][g] + tf
    out = pwb[...]
    for m in range(dm):
        aL = None
        aF = [None] * G
        for k in range(K):
            wl = dwlv[m * K + k:m * K + k + 1, :]
            tl = fXl[k] * wl
            aL = tl if aL is None else aL + tl
            for g in range(G):
                wf = dwfv[m * K + k:m * K + k + 1, g * gw:(g + 1) * gw]
                tf = fXf[k][g] * wf
                aF[g] = tf if aF[g] is None else aF[g] + tf
        out = out + jnp.dot(aL, pwl[m * Cmid:(m + 1) * Cmid, :])
        for g in range(G):
            out = out + jnp.dot(
                aF[g], pwf[m * Cin + g * gw:m * Cin + (g + 1) * gw, :])
    return _elu(out)


def _make_head(K, Pb, Np, Cmid, KK):
    def body(rep_ref, pr_ref, d1w, d1b, d2w, d2b, x0w, x0b,
             x1w, x1b, x2w, x2b, idx_o, lift_o, x_o):
        b = pl.program_id(0)
        rep = rep_ref[0]
        idxs, nbs = _head_compute(rep, pr_ref[0], K, Pb, Np)
        idx_km = jnp.concatenate(idxs, axis=0) + b * jnp.int32(Np)
        idx_o[0, 0] = idx_km
        nb_km = jnp.concatenate(nbs, axis=0)
        rep_t = jnp.concatenate([rep] * K, axis=0)
        plk = nb_km - rep_t
        lift, X = _lift_and_x(
            plk, (d1w, d1b, d2w, d2b, x0w, x0b, x1w, x1b, x2w, x2b), K, Pb)
        lift_o[0, 0] = lift
        x_o[0] = X
    return body


def _make_mix(K, Pb, Cmid, Cin, dm, G, gw, cl, last, onehot_np=0):
    nf = 2 if onehot_np else G

    def body(*refs):
        x_ref, lift_ref = refs[0], refs[1]
        fts_refs = refs[2:2 + nf]
        dwl, dwf, pwl, pwf, pwb = refs[2 + nf:7 + nf]
        if cl:
            fw1, fw2, fb, skip_ref, out_ref = refs[7 + nf:]
        else:
            (out_ref,) = refs[7 + nf:]
        X = x_ref[0]
        lift_sl = lambda l: lift_ref[0, 0, l * Pb:(l + 1) * Pb, :]
        if onehot_np:
            idx_ref, table_ref = fts_refs
            table = table_ref[0]
            b = pl.program_id(0)
            iota_n = jax.lax.broadcasted_iota(
                jnp.int32, (K * Pb, onehot_np), 1)
            ikm = idx_ref[0, 0] - b * jnp.int32(onehot_np)
            oh = (iota_n == ikm).astype(jnp.float32)
            gathered = jax.lax.dot(oh, table,
                                   precision=jax.lax.Precision.HIGHEST,
                                   preferred_element_type=jnp.float32)
            fts_sl = lambda l: [gathered[l * Pb:(l + 1) * Pb, :]]
        else:
            fts_sl = lambda l: [fr[0, 0, l * Pb:(l + 1) * Pb, :]
                                for fr in fts_refs]
        out = _mix_compute(X, lift_sl, fts_sl, dwl, dwf, pwl, pwf, pwb,
                           K, Pb, Cmid, Cin, dm, G, gw)
        if cl:
            out = jnp.dot(out, fw1[...]) + jnp.dot(skip_ref[0], fw2[...]) \
                + fb[...]
            if not last:
                out = _elu(out)
        out_ref[0] = out
    return body


def _make_level1(K, Pb, Np, Cmid, KK, dm, co):
    def body(rep_ref, pr_ref, d1w, d1b, d2w, d2b, x0w, x0b,
             x1w, x1b, x2w, x2b, dwl, dwf, pwl, pwf, pwb, out_ref):
        rep = rep_ref[0]
        _, nbs = _head_compute(rep, pr_ref[0], K, Pb, Np)
        nb_km = jnp.concatenate(nbs, axis=0)
        rep_t = jnp.concatenate([rep] * K, axis=0)
        plk = nb_km - rep_t
        lift, X = _lift_and_x(
            plk, (d1w, d1b, d2w, d2b, x0w, x0b, x1w, x1b, x2w, x2b), K, Pb)
        lift_sl = lambda l: lift[l * Pb:(l + 1) * Pb, :]
        fts_sl = lambda l: [nbs[l]]
        out_ref[0] = _mix_compute(X, lift_sl, fts_sl, dwl, dwf, pwl, pwf,
                                  pwb, K, Pb, Cmid, 3, dm, 1, 3)
    return body


def _gather_rows(table, idx_flat):
    w = 128
    M = idx_flat.shape[1]
    D = table.shape[1]
    mesh = plsc.VectorSubcoreMesh(core_axis_name="core",
                                  subcore_axis_name="subcore")

    @functools.partial(
        pl.kernel,
        out_type=jax.ShapeDtypeStruct((M, D), table.dtype),
        mesh=mesh)
    def _k(x_hbm, i_hbm, o_hbm):
        def gbody(i_vmem, o_vmem):
            pltpu.sync_copy(x_hbm.at[i_vmem.at[0]], o_vmem)

        pltpu.emit_pipeline(
            gbody,
            grid=(M // w,),
            in_specs=[pl.BlockSpec((1, w), index_map=lambda i: (0, i))],
            out_specs=[pl.BlockSpec((w, D), index_map=lambda i: (i, 0))],
            core_axis_name=("core", "subcore"),
            dimension_semantics=(pltpu.PARALLEL,),
        )(i_hbm, o_hbm)

    return _k(table, idx_flat)


def _full(shape):
    n = len(shape)
    return pl.BlockSpec(shape, lambda b, i, _n=n: (0,) * _n)


def _prep_weights(p, K, Cmid, Cin, dm, co):
    dww = p["dw_w"]
    dwl = dww[:, :Cmid, :].transpose(2, 0, 1).reshape(dm * K, Cmid)
    dwf = dww[:, Cmid:, :].transpose(2, 0, 1).reshape(dm * K, Cin)
    pw3 = p["pw_w"].reshape(Cmid + Cin, dm, co)
    pwl = pw3[:Cmid].transpose(1, 0, 2).reshape(dm * Cmid, co)
    pwf = pw3[Cmid:].transpose(1, 0, 2).reshape(dm * Cin, co)
    return dwl, dwf, pwl, pwf


def _xconv(p, rep, pts, fts, K, ci, co, fuse=None, skip_fts=None,
           last=False):
    B, P = rep.shape[0], rep.shape[1]
    Np = pts.shape[1]
    Cmid = co // 4
    dm = min(-(-co // ci), 4)
    KK = K * K
    Pb = _pb_for(P, ci)
    nPb = P // Pb
    KPb = K * Pb
    p3 = jnp.transpose(pts, (0, 2, 1))
    dwl, dwf, pwl, pwf = _prep_weights(p, K, Cmid, ci, dm, co)
    biases = dict(
        d1b=p["d1_b"].reshape(1, -1), d2b=p["d2_b"].reshape(1, -1),
        x0b=p["x0_b"].reshape(1, -1), x1b=p["x1_b"].reshape(1, -1),
        x2b=p["x2_b"].reshape(1, -1), pwb=p["pw_b"].reshape(1, -1))

    head_in = [rep, p3, p["d1_w"], biases["d1b"], p["d2_w"],
               biases["d2b"], p["x0_w"], biases["x0b"], p["x1_w"],
               biases["x1b"], p["x2_w"], biases["x2b"]]
    head_specs = [
        pl.BlockSpec((1, Pb, 3), lambda b, i: (b, i, 0)),
        pl.BlockSpec((1, 3, Np), lambda b, i: (b, 0, 0)),
    ] + [_full(a.shape) for a in head_in[2:]]

    if ci == 3:
        body = _make_level1(K, Pb, Np, Cmid, KK, dm, co)
        out = pl.pallas_call(
            body,
            grid=(B, nPb),
            in_specs=head_specs + [_full(a.shape)
                                   for a in (dwl, dwf, pwl, pwf,
                                             biases["pwb"])],
            out_specs=pl.BlockSpec((1, Pb, co), lambda b, i: (b, i, 0)),
            out_shape=jax.ShapeDtypeStruct((B, P, co), jnp.float32),
        )(*head_in, dwl, dwf, pwl, pwf, biases["pwb"])
        return out

    idx_g, lift, X = pl.pallas_call(
        _make_head(K, Pb, Np, Cmid, KK),
        grid=(B, nPb),
        in_specs=head_specs,
        out_specs=[
            pl.BlockSpec((1, 1, KPb, 1), lambda b, i: (b, i, 0, 0)),
            pl.BlockSpec((1, 1, KPb, Cmid), lambda b, i: (b, i, 0, 0)),
            pl.BlockSpec((1, Pb, KK), lambda b, i: (b, i, 0)),
        ],
        out_shape=[
            jax.ShapeDtypeStruct((B, nPb, KPb, 1), jnp.int32),
            jax.ShapeDtypeStruct((B, nPb, KPb, Cmid), jnp.float32),
            jax.ShapeDtypeStruct((B, P, KK), jnp.float32),
        ],
    )(*head_in)

    use_sc = Np >= 1024
    if use_sc:
        gw = min(ci, 256)
        G = ci // gw
        onehot_np = 0
        table = fts.reshape(B * Np, ci)
        idx_flat = idx_g.reshape(1, B * nPb * KPb)
        fts_in = [
            _gather_rows(table[:, g * gw:(g + 1) * gw],
                         idx_flat).reshape(B, nPb, KPb, gw)
            for g in range(G)
        ]
        fts_specs = [pl.BlockSpec((1, 1, KPb, gw), lambda b, i: (b, i, 0, 0))
                     for _ in range(G)]
    else:
        gw, G = ci, 1
        onehot_np = Np
        fts_in = [idx_g, fts]
        fts_specs = [
            pl.BlockSpec((1, 1, KPb, 1), lambda b, i: (b, i, 0, 0)),
            pl.BlockSpec((1, Np, ci), lambda b, i: (b, 0, 0)),
        ]

    mix_in = ([X, lift] + fts_in
              + [dwl, dwf, pwl, pwf, biases["pwb"]])
    mix_specs = [
        pl.BlockSpec((1, Pb, KK), lambda b, i: (b, i, 0)),
        pl.BlockSpec((1, 1, KPb, Cmid), lambda b, i: (b, i, 0, 0)),
    ] + fts_specs \
      + [_full(a.shape) for a in mix_in[2 + len(fts_in):]]
    if fuse is not None:
        fw1 = fuse["fuse_w"][:co]
        fw2 = fuse["fuse_w"][co:]
        fb = fuse["fuse_b"].reshape(1, -1)
        cl = fw2.shape[0]
        mix_in += [fw1, fw2, fb, skip_fts]
        mix_specs += [_full(fw1.shape), _full(fw2.shape), _full(fb.shape),
                      pl.BlockSpec((1, Pb, cl), lambda b, i: (b, i, 0))]
    else:
        cl = 0

    out = pl.pallas_call(
        _make_mix(K, Pb, Cmid, ci, dm, G, gw, cl, last, onehot_np),
        grid=(B, nPb),
        in_specs=mix_specs,
        out_specs=pl.BlockSpec((1, Pb, co), lambda b, i: (b, i, 0)),
        out_shape=jax.ShapeDtypeStruct((B, P, co), jnp.float32),
    )(*mix_in)
    return out


def kernel(x, params):
    pts = jnp.transpose(x, (0, 2, 1))
    cur_pts, cur_fts = pts, pts
    levels = []
    for p, (ci, co, K, P) in zip(params["enc"], _ENC_CFG):
        rep = cur_pts if P == -1 else cur_pts[:, :P]
        out = _xconv(p, rep, cur_pts, cur_fts, K, ci, co)
        levels.append((rep, out))
        cur_pts, cur_fts = rep, out
    d_pts, d_fts = levels[3]
    skips = [levels[3], levels[2], levels[1], levels[0]]
    n_dec = len(_DEC_CFG)
    for li, (p, (skip_pts, skip_fts), (ci, co, cl, K)) in enumerate(
            zip(params["dec"], skips, _DEC_CFG)):
        d_fts = _xconv(p["xconv"], skip_pts, d_pts, d_fts, K, ci, co,
                       fuse=p, skip_fts=skip_fts, last=(li == n_dec - 1))
        d_pts = skip_pts
    return jnp.transpose(d_fts, (0, 2, 1))


# speedup vs baseline: 7.5158x; 1.0685x over previous
import functools

import jax
import jax.numpy as jnp
from jax.experimental import pallas as pl
from jax.experimental.pallas import tpu as pltpu
from jax.experimental.pallas import tpu_sc as plsc

_PART = 50
_B, _N = 8, 2048
_ENC_CFG = [(3, 256, 8, -1), (256, 256, 12, 768), (256, 512, 16, 384),
            (512, 1024, 16, 128)]
_DEC_CFG = [(1024, 1024, 1024, 16), (1024, 512, 512, 16), (512, 256, 256, 12),
            (256, _PART, 256, 8)]


def _elu(x):
    return jnp.where(x > 0, x, jnp.exp(jnp.minimum(x, 0.0)) - 1.0)


def _pb_for(p, ci):
    if ci >= 1024:
        return 64
    return 256 if p % 256 == 0 else 128


def _head_compute(rep, p3, K, Pb, Np):
    rx, ry, rz = rep[:, 0:1], rep[:, 1:2], rep[:, 2:3]
    px, py, pz = p3[0:1, :], p3[1:2, :], p3[2:3, :]
    rep2 = rx * rx + ry * ry + rz * rz
    pts2 = px * px + py * py + pz * pz
    cross = jnp.dot(rep, p3, preferred_element_type=jnp.float32)
    d = rep2 + pts2 - 2.0 * cross
    iota = jax.lax.broadcasted_iota(jnp.int32, (Pb, Np), 1)
    big = jnp.float32(3.0e38)
    idxs, nbs = [], []
    for _ in range(K):
        mn = jnp.min(d, axis=1, keepdims=True)
        cand = jnp.where(d <= mn, iota, jnp.int32(Np))
        ij = jnp.min(cand, axis=1, keepdims=True)
        mask = iota == ij
        mf = mask.astype(jnp.float32)
        sx = jnp.sum(mf * px, axis=1, keepdims=True)
        sy = jnp.sum(mf * py, axis=1, keepdims=True)
        sz = jnp.sum(mf * pz, axis=1, keepdims=True)
        nb = jnp.concatenate([sx, sy, sz], axis=1)
        d = jnp.where(mask, big, d)
        idxs.append(ij)
        nbs.append(nb)
    return idxs, nbs


def _lift_and_x(plk, refs, K, Pb):
    (d1w, d1b, d2w, d2b, x0w, x0b, x1w, x1b, x2w, x2b) = refs
    lift = _elu(jnp.dot(plk, d1w[...]) + d1b[...])
    lift = _elu(jnp.dot(lift, d2w[...]) + d2b[...])
    acc = None
    for k in range(K):
        t = jnp.dot(plk[k * Pb:(k + 1) * Pb, :], x0w[k])
        acc = t if acc is None else acc + t
    X = _elu(acc + x0b[...])
    X = _elu(jnp.dot(X, x1w[...]) + x1b[...])
    X = jnp.dot(X, x2w[...]) + x2b[...]
    return lift, X


def _treesum(xs):
    xs = list(xs)
    while len(xs) > 1:
        nxt = [xs[i] + xs[i + 1] for i in range(0, len(xs) - 1, 2)]
        if len(xs) % 2:
            nxt.append(xs[-1])
        xs = nxt
    return xs[0]


def _mix_compute(X, lift_sl, fts_sl, dwl, dwf, pwl, pwf, pwb,
                 K, Pb, Cmid, Cin, dm, G, gw):
    dwlv, dwfv = dwl[...], dwf[...]
    lvs = [lift_sl(l) for l in range(K)]
    fvs = [fts_sl(l) for l in range(K)]
    fXl = []
    fXf = []
    for k in range(K):
        xk = [X[:, k * K + l:k * K + l + 1] for l in range(K)]
        fXl.append(_treesum(xk[l] * lvs[l] for l in range(K)))
        fXf.append([_treesum(xk[l] * fvs[l][g] for l in range(K))
                    for g in range(G)])
    out = pwb[...]
    for m in range(dm):
        aL = _treesum(
            fXl[k] * dwlv[m * K + k:m * K + k + 1, :] for k in range(K))
        out = out + jnp.dot(aL, pwl[m * Cmid:(m + 1) * Cmid, :])
        for g in range(G):
            aF = _treesum(
                fXf[k][g] * dwfv[m * K + k:m * K + k + 1,
                                 g * gw:(g + 1) * gw]
                for k in range(K))
            out = out + jnp.dot(
                aF, pwf[m * Cin + g * gw:m * Cin + (g + 1) * gw, :])
    return _elu(out)


def _make_head(K, Pb, Np, Cmid, KK):
    def body(rep_ref, pr_ref, d1w, d1b, d2w, d2b, x0w, x0b,
             x1w, x1b, x2w, x2b, idx_o, lift_o, x_o):
        b = pl.program_id(0)
        rep = rep_ref[0]
        idxs, nbs = _head_compute(rep, pr_ref[0], K, Pb, Np)
        idx_km = jnp.concatenate(idxs, axis=0) + b * jnp.int32(Np)
        idx_o[0, 0] = idx_km
        nb_km = jnp.concatenate(nbs, axis=0)
        rep_t = jnp.concatenate([rep] * K, axis=0)
        plk = nb_km - rep_t
        lift, X = _lift_and_x(
            plk, (d1w, d1b, d2w, d2b, x0w, x0b, x1w, x1b, x2w, x2b), K, Pb)
        lift_o[0, 0] = lift
        x_o[0] = X
    return body


def _make_mix(K, Pb, Cmid, Cin, dm, G, gw, cl, last, onehot_np=0):
    nf = 2 if onehot_np else G

    def body(*refs):
        x_ref, lift_ref = refs[0], refs[1]
        fts_refs = refs[2:2 + nf]
        dwl, dwf, pwl, pwf, pwb = refs[2 + nf:7 + nf]
        if cl:
            fw1, fw2, fb, skip_ref, out_ref = refs[7 + nf:]
        else:
            (out_ref,) = refs[7 + nf:]
        X = x_ref[0]
        lift_sl = lambda l: lift_ref[0, 0, l * Pb:(l + 1) * Pb, :]
        if onehot_np:
            idx_ref, table_ref = fts_refs
            table = table_ref[0]
            b = pl.program_id(0)
            iota_n = jax.lax.broadcasted_iota(
                jnp.int32, (K * Pb, onehot_np), 1)
            ikm = idx_ref[0, 0] - b * jnp.int32(onehot_np)
            oh = (iota_n == ikm).astype(jnp.bfloat16)
            t1 = table.astype(jnp.bfloat16)
            r1 = table - t1.astype(jnp.float32)
            t2 = r1.astype(jnp.bfloat16)
            t3 = (r1 - t2.astype(jnp.float32)).astype(jnp.bfloat16)
            gathered = (
                jnp.dot(oh, t1, preferred_element_type=jnp.float32)
                + jnp.dot(oh, t2, preferred_element_type=jnp.float32)
                + jnp.dot(oh, t3, preferred_element_type=jnp.float32))
            fts_sl = lambda l: [gathered[l * Pb:(l + 1) * Pb, :]]
        else:
            fts_sl = lambda l: [fr[0, 0, l * Pb:(l + 1) * Pb, :]
                                for fr in fts_refs]
        out = _mix_compute(X, lift_sl, fts_sl, dwl, dwf, pwl, pwf, pwb,
                           K, Pb, Cmid, Cin, dm, G, gw)
        if cl:
            out = jnp.dot(out, fw1[...]) + jnp.dot(skip_ref[0], fw2[...]) \
                + fb[...]
            if not last:
                out = _elu(out)
        out_ref[0] = out
    return body


def _make_level1(K, Pb, Np, Cmid, KK, dm, co):
    def body(rep_ref, pr_ref, d1w, d1b, d2w, d2b, x0w, x0b,
             x1w, x1b, x2w, x2b, dwl, dwf, pwl, pwf, pwb, out_ref):
        rep = rep_ref[0]
        _, nbs = _head_compute(rep, pr_ref[0], K, Pb, Np)
        nb_km = jnp.concatenate(nbs, axis=0)
        rep_t = jnp.concatenate([rep] * K, axis=0)
        plk = nb_km - rep_t
        lift, X = _lift_and_x(
            plk, (d1w, d1b, d2w, d2b, x0w, x0b, x1w, x1b, x2w, x2b), K, Pb)
        lift_sl = lambda l: lift[l * Pb:(l + 1) * Pb, :]
        fts_sl = lambda l: [nbs[l]]
        out_ref[0] = _mix_compute(X, lift_sl, fts_sl, dwl, dwf, pwl, pwf,
                                  pwb, K, Pb, Cmid, 3, dm, 1, 3)
    return body


def _gather_rows(table, idx_flat):
    w = 128
    M = idx_flat.shape[1]
    D = table.shape[1]
    mesh = plsc.VectorSubcoreMesh(core_axis_name="core",
                                  subcore_axis_name="subcore")

    @functools.partial(
        pl.kernel,
        out_type=jax.ShapeDtypeStruct((M, D), table.dtype),
        mesh=mesh)
    def _k(x_hbm, i_hbm, o_hbm):
        def gbody(i_vmem, o_vmem):
            pltpu.sync_copy(x_hbm.at[i_vmem.at[0]], o_vmem)

        pltpu.emit_pipeline(
            gbody,
            grid=(M // w,),
            in_specs=[pl.BlockSpec((1, w), index_map=lambda i: (0, i))],
            out_specs=[pl.BlockSpec((w, D), index_map=lambda i: (i, 0))],
            core_axis_name=("core", "subcore"),
            dimension_semantics=(pltpu.PARALLEL,),
        )(i_hbm, o_hbm)

    return _k(table, idx_flat)


def _full(shape):
    n = len(shape)
    return pl.BlockSpec(shape, lambda b, i, _n=n: (0,) * _n)


def _prep_weights(p, K, Cmid, Cin, dm, co):
    dww = p["dw_w"]
    dwl = dww[:, :Cmid, :].transpose(2, 0, 1).reshape(dm * K, Cmid)
    dwf = dww[:, Cmid:, :].transpose(2, 0, 1).reshape(dm * K, Cin)
    pw3 = p["pw_w"].reshape(Cmid + Cin, dm, co)
    pwl = pw3[:Cmid].transpose(1, 0, 2).reshape(dm * Cmid, co)
    pwf = pw3[Cmid:].transpose(1, 0, 2).reshape(dm * Cin, co)
    return dwl, dwf, pwl, pwf


def _xconv(p, rep, pts, fts, K, ci, co, fuse=None, skip_fts=None,
           last=False):
    B, P = rep.shape[0], rep.shape[1]
    Np = pts.shape[1]
    Cmid = co // 4
    dm = min(-(-co // ci), 4)
    KK = K * K
    Pb = _pb_for(P, ci)
    nPb = P // Pb
    KPb = K * Pb
    p3 = jnp.transpose(pts, (0, 2, 1))
    dwl, dwf, pwl, pwf = _prep_weights(p, K, Cmid, ci, dm, co)
    biases = dict(
        d1b=p["d1_b"].reshape(1, -1), d2b=p["d2_b"].reshape(1, -1),
        x0b=p["x0_b"].reshape(1, -1), x1b=p["x1_b"].reshape(1, -1),
        x2b=p["x2_b"].reshape(1, -1), pwb=p["pw_b"].reshape(1, -1))

    head_in = [rep, p3, p["d1_w"], biases["d1b"], p["d2_w"],
               biases["d2b"], p["x0_w"], biases["x0b"], p["x1_w"],
               biases["x1b"], p["x2_w"], biases["x2b"]]
    head_specs = [
        pl.BlockSpec((1, Pb, 3), lambda b, i: (b, i, 0)),
        pl.BlockSpec((1, 3, Np), lambda b, i: (b, 0, 0)),
    ] + [_full(a.shape) for a in head_in[2:]]

    if ci == 3:
        body = _make_level1(K, Pb, Np, Cmid, KK, dm, co)
        out = pl.pallas_call(
            body,
            grid=(B, nPb),
            in_specs=head_specs + [_full(a.shape)
                                   for a in (dwl, dwf, pwl, pwf,
                                             biases["pwb"])],
            out_specs=pl.BlockSpec((1, Pb, co), lambda b, i: (b, i, 0)),
            out_shape=jax.ShapeDtypeStruct((B, P, co), jnp.float32),
        )(*head_in, dwl, dwf, pwl, pwf, biases["pwb"])
        return out

    idx_g, lift, X = pl.pallas_call(
        _make_head(K, Pb, Np, Cmid, KK),
        grid=(B, nPb),
        in_specs=head_specs,
        out_specs=[
            pl.BlockSpec((1, 1, KPb, 1), lambda b, i: (b, i, 0, 0)),
            pl.BlockSpec((1, 1, KPb, Cmid), lambda b, i: (b, i, 0, 0)),
            pl.BlockSpec((1, Pb, KK), lambda b, i: (b, i, 0)),
        ],
        out_shape=[
            jax.ShapeDtypeStruct((B, nPb, KPb, 1), jnp.int32),
            jax.ShapeDtypeStruct((B, nPb, KPb, Cmid), jnp.float32),
            jax.ShapeDtypeStruct((B, P, KK), jnp.float32),
        ],
    )(*head_in)

    use_sc = Np >= 1024
    if use_sc:
        gw = min(ci, 256)
        G = ci // gw
        onehot_np = 0
        table = fts.reshape(B * Np, ci)
        idx_flat = idx_g.reshape(1, B * nPb * KPb)
        fts_in = [
            _gather_rows(table[:, g * gw:(g + 1) * gw],
                         idx_flat).reshape(B, nPb, KPb, gw)
            for g in range(G)
        ]
        fts_specs = [pl.BlockSpec((1, 1, KPb, gw), lambda b, i: (b, i, 0, 0))
                     for _ in range(G)]
    else:
        gw, G = ci, 1
        onehot_np = Np
        fts_in = [idx_g, fts]
        fts_specs = [
            pl.BlockSpec((1, 1, KPb, 1), lambda b, i: (b, i, 0, 0)),
            pl.BlockSpec((1, Np, ci), lambda b, i: (b, 0, 0)),
        ]

    mix_in = ([X, lift] + fts_in
              + [dwl, dwf, pwl, pwf, biases["pwb"]])
    mix_specs = [
        pl.BlockSpec((1, Pb, KK), lambda b, i: (b, i, 0)),
        pl.BlockSpec((1, 1, KPb, Cmid), lambda b, i: (b, i, 0, 0)),
    ] + fts_specs \
      + [_full(a.shape) for a in mix_in[2 + len(fts_in):]]
    if fuse is not None:
        fw1 = fuse["fuse_w"][:co]
        fw2 = fuse["fuse_w"][co:]
        fb = fuse["fuse_b"].reshape(1, -1)
        cl = fw2.shape[0]
        mix_in += [fw1, fw2, fb, skip_fts]
        mix_specs += [_full(fw1.shape), _full(fw2.shape), _full(fb.shape),
                      pl.BlockSpec((1, Pb, cl), lambda b, i: (b, i, 0))]
    else:
        cl = 0

    out = pl.pallas_call(
        _make_mix(K, Pb, Cmid, ci, dm, G, gw, cl, last, onehot_np),
        grid=(B, nPb),
        in_specs=mix_specs,
        out_specs=pl.BlockSpec((1, Pb, co), lambda b, i: (b, i, 0)),
        out_shape=jax.ShapeDtypeStruct((B, P, co), jnp.float32),
    )(*mix_in)
    return out


def kernel(x, params):
    pts = jnp.transpose(x, (0, 2, 1))
    cur_pts, cur_fts = pts, pts
    levels = []
    for p, (ci, co, K, P) in zip(params["enc"], _ENC_CFG):
        rep = cur_pts if P == -1 else cur_pts[:, :P]
        out = _xconv(p, rep, cur_pts, cur_fts, K, ci, co)
        levels.append((rep, out))
        cur_pts, cur_fts = rep, out
    d_pts, d_fts = levels[3]
    skips = [levels[3], levels[2], levels[1], levels[0]]
    n_dec = len(_DEC_CFG)
    for li, (p, (skip_pts, skip_fts), (ci, co, cl, K)) in enumerate(
            zip(params["dec"], skips, _DEC_CFG)):
        d_fts = _xconv(p["xconv"], skip_pts, d_pts, d_fts, K, ci, co,
                       fuse=p, skip_fts=skip_fts, last=(li == n_dec - 1))
        d_pts = skip_pts
    return jnp.transpose(d_fts, (0, 2, 1))
